# baseline (speedup 1.0000x reference)
import functools

import jax
import jax.numpy as jnp
from jax import lax
from jax.experimental import pallas as pl
from jax.experimental.pallas import tpu as pltpu
from jax.experimental.pallas import tpu_sc as plsc

NH = 128
NNODE = 50000
NE = 160000
BLK = 5000


def _layernorm(y, g, beta):
    mu = jnp.mean(y, axis=-1, keepdims=True)
    d = y - mu
    var = jnp.mean(d * d, axis=-1, keepdims=True)
    return d / jnp.sqrt(var + 1e-5) * g + beta


def _mlp_body(x_ref, w1_ref, b1_ref, w2_ref, g_ref, beta_ref, *refs,
              residual, nproj):
    pw = refs[:nproj]
    o_ref = refs[nproj]
    po = refs[nproj + 1:]
    x = x_ref[0]
    h = jnp.dot(x, w1_ref[...], preferred_element_type=jnp.float32) + b1_ref[...]
    h = h * jax.nn.sigmoid(h)
    y = jnp.dot(h, w2_ref[...], preferred_element_type=jnp.float32)
    out = _layernorm(y, g_ref[...], beta_ref[...])
    if residual:
        out = out + x
    o_ref[...] = out
    for k in range(nproj):
        po[k][...] = jnp.dot(out, pw[k][...], preferred_element_type=jnp.float32)


def _mlp_call(x, mp, residual=False, proj=()):
    if x.ndim == 2:
        x = x[None]
    _, n, f = x.shape
    w1 = mp['W1']
    spec = pl.BlockSpec((BLK, NH), lambda i: (i, 0))
    wspec = pl.BlockSpec((NH, NH), lambda i: (0, 0))
    vspec = pl.BlockSpec((1, NH), lambda i: (0, 0))
    nproj = len(proj)
    out = pl.pallas_call(
        functools.partial(_mlp_body, residual=residual, nproj=nproj),
        grid=(n // BLK,),
        in_specs=[pl.BlockSpec((1, BLK, f), lambda i: (0, i, 0)),
                  pl.BlockSpec((f, NH), lambda i: (0, 0)),
                  vspec, wspec, vspec, vspec] + [wspec] * nproj,
        out_specs=[spec] * (1 + nproj),
        out_shape=[jax.ShapeDtypeStruct((n, NH), jnp.float32)] * (1 + nproj),
    )(x, w1, mp['b1'].reshape(1, NH), mp['W2'], mp['g'].reshape(1, NH),
      mp['beta'].reshape(1, NH), *proj)
    return out if nproj else out[0]


def _edge_body(ea_ref, gs_ref, gr_ref, w1_ref, b1_ref, w2_ref, g_ref, beta_ref,
               *o_refs, with_res):
    ea = ea_ref[...]
    h = (jnp.dot(ea, w1_ref[...], preferred_element_type=jnp.float32)
         + gs_ref[...] + gr_ref[...] + b1_ref[...])
    h = h * jax.nn.sigmoid(h)
    y = jnp.dot(h, w2_ref[...], preferred_element_type=jnp.float32)
    eu = _layernorm(y, g_ref[...], beta_ref[...])
    o_refs[0][...] = eu
    if with_res:
        o_refs[1][...] = ea + eu


def _edge_call(ea, gs, gr, w1e, ep, with_res):
    spec = pl.BlockSpec((BLK, NH), lambda i: (i, 0))
    wspec = pl.BlockSpec((NH, NH), lambda i: (0, 0))
    vspec = pl.BlockSpec((1, NH), lambda i: (0, 0))
    out_specs = [spec] + ([spec] if with_res else [])
    out_shape = [jax.ShapeDtypeStruct((NE, NH), jnp.float32)]
    if with_res:
        out_shape.append(jax.ShapeDtypeStruct((NE, NH), jnp.float32))
    out = pl.pallas_call(
        functools.partial(_edge_body, with_res=with_res),
        grid=(NE // BLK,),
        in_specs=[spec, spec, spec, wspec, vspec, wspec, vspec, vspec],
        out_specs=out_specs,
        out_shape=out_shape,
    )(ea, gs, gr, w1e, ep['b1'].reshape(1, NH), ep['W2'],
      ep['g'].reshape(1, NH), ep['beta'].reshape(1, NH))
    return out if with_res else (out[0], None)


def _recv_body(xr_ref, sums_ref, cnt_ref, w1u_ref, w1v_ref, b1_ref, w2_ref,
               g_ref, beta_ref, *refs, nproj):
    pw = refs[:nproj]
    o_ref = refs[nproj]
    po = refs[nproj + 1:]
    xr = xr_ref[...]
    cnt = cnt_ref[0]
    c = cnt[0, :, 0:1] + cnt[1, :, 0:1]
    coll = sums_ref[...] / jnp.maximum(c, 1.0)
    h = (jnp.dot(xr, w1u_ref[...], preferred_element_type=jnp.float32)
         + jnp.dot(coll, w1v_ref[...], preferred_element_type=jnp.float32)
         + b1_ref[...])
    h = h * jax.nn.sigmoid(h)
    y = jnp.dot(h, w2_ref[...], preferred_element_type=jnp.float32)
    out = xr + _layernorm(y, g_ref[...], beta_ref[...])
    o_ref[...] = out
    for k in range(nproj):
        po[k][...] = jnp.dot(out, pw[k][...], preferred_element_type=jnp.float32)


def _recv_call(xr, sums, cnt, cset, rp, proj=()):
    spec = pl.BlockSpec((BLK, NH), lambda i: (i, 0))
    wspec = pl.BlockSpec((NH, NH), lambda i: (0, 0))
    vspec = pl.BlockSpec((1, NH), lambda i: (0, 0))
    w1 = rp['W1']
    nproj = len(proj)
    out = pl.pallas_call(
        functools.partial(_recv_body, nproj=nproj),
        grid=(NNODE // BLK,),
        in_specs=[spec, spec,
                  pl.BlockSpec((1, 2, BLK, 16), lambda i: (cset, 0, i, 0)),
                  wspec, wspec, vspec, wspec, vspec, vspec] + [wspec] * nproj,
        out_specs=[spec] * (1 + nproj),
        out_shape=[jax.ShapeDtypeStruct((NNODE, NH), jnp.float32)] * (1 + nproj),
    )(xr, sums, cnt, w1[:NH], w1[NH:], rp['b1'].reshape(1, NH), rp['W2'],
      rp['g'].reshape(1, NH), rp['beta'].reshape(1, NH), *proj)
    return out if nproj else out[0]


@functools.lru_cache(maxsize=None)
def _sc_mesh():
    return plsc.VectorSubcoreMesh(core_axis_name="c", subcore_axis_name="s",
                                  num_cores=2, num_subcores=16)


NW = 32
GCH = 128
G_PAD = 163840
G_PER_W = G_PAD // NW
G_NCH = G_PER_W // GCH

S_CH = 128
S_NCHUNK = NE // S_CH
Z_ROWS = 1000
C_PER_CORE = S_NCHUNK // 2


def _gather_body(ps_hbm, pr_hbm, idxs_hbm, idxr_hbm, gs_hbm, gr_hbm,
                 idxs_v, idxr_v, rs_a, rr_a, rs_b, rr_b,
                 sem_sa, sem_ra, sem_sb, sem_rb):
    wid = lax.axis_index("s") * 2 + lax.axis_index("c")
    base = pl.multiple_of(wid * G_PER_W, 8)
    pltpu.sync_copy(idxs_hbm.at[pl.ds(base, G_PER_W)], idxs_v)
    pltpu.sync_copy(idxr_hbm.at[pl.ds(base, G_PER_W)], idxr_v)

    def fire(ci, rs_v, rr_v, sem_s, sem_r):
        o = pl.multiple_of(ci * GCH, 8)
        cs = pltpu.async_copy(ps_hbm.at[idxs_v.at[pl.ds(o, GCH)]], rs_v, sem_s)
        cr = pltpu.async_copy(pr_hbm.at[idxr_v.at[pl.ds(o, GCH)]], rr_v, sem_r)
        return cs, cr

    def drain(ci, rs_v, rr_v, sem_s, sem_r):
        o = pl.multiple_of(ci * GCH, 8)
        pltpu.make_async_copy(ps_hbm.at[idxs_v.at[pl.ds(o, GCH)]], rs_v,
                              sem_s).wait()
        pltpu.make_async_copy(pr_hbm.at[idxr_v.at[pl.ds(o, GCH)]], rr_v,
                              sem_r).wait()
        off = pl.multiple_of(base + ci * GCH, 8)
        pltpu.sync_copy(rs_v, gs_hbm.at[pl.ds(off, GCH)])
        pltpu.sync_copy(rr_v, gr_hbm.at[pl.ds(off, GCH)])

    fire(0, rs_a, rr_a, sem_sa, sem_ra)

    def body(k, carry):
        @pl.when(k + 1 < G_NCH)
        def _():
            lax.cond(lax.rem(k, 2) == 0,
                     lambda: (fire(k + 1, rs_b, rr_b, sem_sb, sem_rb), None)[1],
                     lambda: (fire(k + 1, rs_a, rr_a, sem_sa, sem_ra), None)[1])

        lax.cond(lax.rem(k, 2) == 0,
                 lambda: (drain(k, rs_a, rr_a, sem_sa, sem_ra), None)[1],
                 lambda: (drain(k, rs_b, rr_b, sem_sb, sem_rb), None)[1])
        return carry

    lax.fori_loop(0, G_NCH, body, 0)


@functools.lru_cache(maxsize=None)
def _gather_kernel():
    return pl.kernel(
        _gather_body,
        out_type=(jax.ShapeDtypeStruct((G_PAD, NH), jnp.float32),) * 2,
        mesh=_sc_mesh(),
        compiler_params=pltpu.CompilerParams(use_tc_tiling_on_sc=False),
        scratch_types=[
            pltpu.VMEM((G_PER_W,), jnp.int32),
            pltpu.VMEM((G_PER_W,), jnp.int32),
            pltpu.VMEM((GCH, NH), jnp.float32),
            pltpu.VMEM((GCH, NH), jnp.float32),
            pltpu.VMEM((GCH, NH), jnp.float32),
            pltpu.VMEM((GCH, NH), jnp.float32),
            pltpu.SemaphoreType.DMA,
            pltpu.SemaphoreType.DMA,
            pltpu.SemaphoreType.DMA,
            pltpu.SemaphoreType.DMA,
        ],
    )


def _gather_call(ps, pr, sg, rg):
    return _gather_kernel()(ps, pr, sg, rg)


def _scatter_body(eu_hbm, idx_hbm, zz_hbm, sums_hbm, idx_v, rows_v, idx_b,
                  rows_b, acc_sh, sem_a, sem_b):
    core = lax.axis_index("c")
    t = lax.axis_index("s")
    nch = jnp.where(t < 2, 79, 78)
    nwb = jnp.where(t < 2, 4, 3)

    for cc in range(2):
        col0 = (core * 2 + cc) * 32

        def zc(j, carry):
            off = pl.multiple_of((t + 16 * j) * Z_ROWS, 8)
            pltpu.sync_copy(zz_hbm, acc_sh.at[pl.ds(off, Z_ROWS)])
            return carry

        lax.fori_loop(0, nwb, zc, 0)
        plsc.subcore_barrier()

        def sfire(k, idx_v, rows_v, sem):
            ch = t + 16 * k
            off = pl.multiple_of(ch * S_CH, 8)
            pltpu.async_copy(idx_hbm.at[pl.ds(off, S_CH)], idx_v, sem)
            pltpu.async_copy(eu_hbm.at[pl.ds(off, S_CH), pl.ds(col0, 32)],
                             rows_v, sem)

        def sdrain(k, idx_v, rows_v, sem):
            ch = t + 16 * k
            off = pl.multiple_of(ch * S_CH, 8)
            pltpu.make_async_copy(idx_hbm.at[pl.ds(off, S_CH)], idx_v,
                                  sem).wait()
            pltpu.make_async_copy(eu_hbm.at[pl.ds(off, S_CH), pl.ds(col0, 32)],
                                  rows_v, sem).wait()
            pltpu.sync_copy(rows_v, acc_sh.at[idx_v], add=True)

        sfire(0, idx_v, rows_v, sem_a)

        def sc_body(k, carry):
            nxt = k + 1

            @pl.when(nxt < nch)
            def _():
                lax.cond(lax.rem(k, 2) == 0,
                         lambda: sfire(nxt, idx_b, rows_b, sem_b),
                         lambda: sfire(nxt, idx_v, rows_v, sem_a))

            lax.cond(lax.rem(k, 2) == 0,
                     lambda: sdrain(k, idx_v, rows_v, sem_a),
                     lambda: sdrain(k, idx_b, rows_b, sem_b))
            return carry

        lax.fori_loop(0, nch, sc_body, 0)
        plsc.subcore_barrier()

        def wb(j, carry):
            off = pl.multiple_of((t + 16 * j) * Z_ROWS, 8)
            pltpu.sync_copy(acc_sh.at[pl.ds(off, Z_ROWS)],
                            sums_hbm.at[pl.ds(off, Z_ROWS), pl.ds(col0, 32)])
            return carry

        lax.fori_loop(0, nwb, wb, 0)
        plsc.subcore_barrier()


@functools.lru_cache(maxsize=None)
def _scatter_kernel():
    return pl.kernel(
        _scatter_body,
        out_type=jax.ShapeDtypeStruct((NNODE, NH), jnp.float32),
        mesh=_sc_mesh(),
        compiler_params=pltpu.CompilerParams(use_tc_tiling_on_sc=False),
        scratch_types=[
            pltpu.VMEM((S_CH,), jnp.int32),
            pltpu.VMEM((S_CH, 32), jnp.float32),
            pltpu.VMEM((S_CH,), jnp.int32),
            pltpu.VMEM((S_CH, 32), jnp.float32),
            pltpu.VMEM_SHARED((NNODE, 32), jnp.float32),
            pltpu.SemaphoreType.DMA,
            pltpu.SemaphoreType.DMA,
        ],
    )


def _scatter_call(eu, rs, zz):
    return _scatter_kernel()(eu, rs, zz)


def _counts_body(i0_hbm, i1_hbm, i2_hbm, ones_hbm, zz_hbm, cnt_hbm, idx_v,
                 ones_v, acc_sh):
    core = lax.axis_index("c")
    t = lax.axis_index("s")
    nch = jnp.where(t < 1, 40, 39)
    nwb = jnp.where(t < 2, 4, 3)

    pltpu.sync_copy(ones_hbm, ones_v)

    for si, idx_hbm in enumerate((i0_hbm, i1_hbm, i2_hbm)):
        def zc(j, carry):
            off = pl.multiple_of((t + 16 * j) * Z_ROWS, 8)
            pltpu.sync_copy(zz_hbm, acc_sh.at[pl.ds(off, Z_ROWS)])
            return carry

        lax.fori_loop(0, nwb, zc, 0)
        plsc.subcore_barrier()

        def cbody(k, carry):
            ch = core * C_PER_CORE + t + 16 * k
            off = pl.multiple_of(ch * S_CH, 8)
            pltpu.sync_copy(idx_hbm.at[pl.ds(off, S_CH)], idx_v)
            pltpu.sync_copy(ones_v, acc_sh.at[idx_v], add=True)
            return carry

        lax.fori_loop(0, nch, cbody, 0)
        plsc.subcore_barrier()

        def wb(j, carry):
            off = pl.multiple_of((t + 16 * j) * Z_ROWS, 8)
            pltpu.sync_copy(acc_sh.at[pl.ds(off, Z_ROWS)],
                            cnt_hbm.at[si, core, pl.ds(off, Z_ROWS)])
            return carry

        lax.fori_loop(0, nwb, wb, 0)
        plsc.subcore_barrier()


@functools.lru_cache(maxsize=None)
def _counts_kernel():
    return pl.kernel(
        _counts_body,
        out_type=jax.ShapeDtypeStruct((3, 2, NNODE, 16), jnp.float32),
        mesh=_sc_mesh(),
        compiler_params=pltpu.CompilerParams(use_tc_tiling_on_sc=False),
        scratch_types=[
            pltpu.VMEM((S_CH,), jnp.int32),
            pltpu.VMEM((S_CH, 16), jnp.float32),
            pltpu.VMEM_SHARED((NNODE, 16), jnp.float32),
        ],
    )


def _counts_call(r0, r1, r2, ones, zz16):
    return _counts_kernel()(r0, r1, r2, ones, zz16)


def _mp_layer(mpp, ea, ps, pr, sg, rg, rs, cnt, cset, xr, with_edge_res,
              zz32, next_proj):
    gs, gr = _gather_call(ps, pr, sg, rg)
    eu, ea2 = _edge_call(ea, gs, gr, mpp['edge']['W1'][:NH], mpp['edge'],
                         with_edge_res)
    sums = _scatter_call(eu, rs, zz32)
    r = _recv_call(xr, sums, cnt, cset, mpp['recv'], proj=next_proj)
    return r, ea2


def kernel(x_grid, x_mesh, g2m_edge_attr, mm_edge_attr, m2g_edge_attr,
           g2m_edge_index, mm_edge_index, m2g_edge_index, params):
    p = params
    xg0 = x_grid
    xm0 = x_mesh
    g2m0 = g2m_edge_attr
    mm0 = mm_edge_attr
    m2g0 = m2g_edge_attr

    enc_w1 = p['encoder']['edge']['W1']
    proc_w1 = [lp['edge']['W1'] for lp in p['processors']]
    dec_w1 = p['decoder']['edge']['W1']

    xg, ps_enc = _mlp_call(xg0, p['embed_grid'], proj=(enc_w1[NH:2 * NH],))
    xm, pr_enc = _mlp_call(xm0, p['embed_mesh'], proj=(enc_w1[2 * NH:],))
    g2m = _mlp_call(g2m0, p['embed_g2m'])
    mm = _mlp_call(mm0, p['embed_mm'])
    m2g = _mlp_call(m2g0, p['embed_m2g'])

    def prep_idx(ei):
        pad = (0, G_PAD - NE)
        return jnp.pad(ei[0], pad), jnp.pad(ei[1], pad), ei[1]

    g2m_sg, g2m_rg, g2m_rs = prep_idx(g2m_edge_index)
    mm_sg, mm_rg, mm_rs = prep_idx(mm_edge_index)
    m2g_sg, m2g_rg, m2g_rs = prep_idx(m2g_edge_index)

    zz32 = jnp.zeros((Z_ROWS, 32), jnp.float32)
    zz16 = jnp.zeros((Z_ROWS, 16), jnp.float32)
    ones16 = jnp.ones((S_CH, 16), jnp.float32)

    cnt3 = _counts_call(g2m_rs, mm_rs, m2g_rs, ones16, zz16)

    r, _ = _mp_layer(p['encoder'], g2m, ps_enc, pr_enc, g2m_sg, g2m_rg,
                     g2m_rs, cnt3, 0, xm, False, zz32,
                     (proc_w1[0][NH:2 * NH], proc_w1[0][2 * NH:]))
    xm, ps_c, pr_c = r
    xg2, pr_dec = _mlp_call(xg, p['encoder']['send'], residual=True,
                            proj=(dec_w1[2 * NH:],))
    ps_dec = None
    for i, lp in enumerate(p['processors']):
        if i < 4:
            nxt = (proc_w1[i + 1][NH:2 * NH], proc_w1[i + 1][2 * NH:])
        else:
            nxt = (dec_w1[NH:2 * NH],)
        r, mm = _mp_layer(lp, mm, ps_c, pr_c, mm_sg, mm_rg, mm_rs, cnt3, 1,
                          xm, True, zz32, nxt)
        if i < 4:
            xm, ps_c, pr_c = r
        else:
            xm, ps_dec = r
    xg_out, _ = _mp_layer(p['decoder'], m2g, ps_dec, pr_dec, m2g_sg, m2g_rg,
                          m2g_rs, cnt3, 2, xg2, False, zz32, ())
    return xg_out[None]


# speedup vs baseline: 1.3994x; 1.3994x over previous
import functools

import jax
import jax.numpy as jnp
from jax import lax
from jax.experimental import pallas as pl
from jax.experimental.pallas import tpu as pltpu
from jax.experimental.pallas import tpu_sc as plsc

NH = 128
NNODE = 50000
NE = 160000
BLK = 5000


def _layernorm(y, g, beta):
    mu = jnp.mean(y, axis=-1, keepdims=True)
    d = y - mu
    var = jnp.mean(d * d, axis=-1, keepdims=True)
    return d / jnp.sqrt(var + 1e-5) * g + beta


def _mlp_body(x_ref, w1_ref, b1_ref, w2_ref, g_ref, beta_ref, *refs,
              residual, nproj):
    pw = refs[:nproj]
    o_ref = refs[nproj]
    po = refs[nproj + 1:]
    x = x_ref[0]
    h = jnp.dot(x, w1_ref[...], preferred_element_type=jnp.float32) + b1_ref[...]
    h = h * jax.nn.sigmoid(h)
    y = jnp.dot(h, w2_ref[...], preferred_element_type=jnp.float32)
    out = _layernorm(y, g_ref[...], beta_ref[...])
    if residual:
        out = out + x
    o_ref[...] = out
    for k in range(nproj):
        po[k][...] = jnp.dot(out, pw[k][...], preferred_element_type=jnp.float32)


def _mlp_call(x, mp, residual=False, proj=()):
    if x.ndim == 2:
        x = x[None]
    _, n, f = x.shape
    w1 = mp['W1']
    spec = pl.BlockSpec((BLK, NH), lambda i: (i, 0))
    wspec = pl.BlockSpec((NH, NH), lambda i: (0, 0))
    vspec = pl.BlockSpec((1, NH), lambda i: (0, 0))
    nproj = len(proj)
    out = pl.pallas_call(
        functools.partial(_mlp_body, residual=residual, nproj=nproj),
        grid=(n // BLK,),
        in_specs=[pl.BlockSpec((1, BLK, f), lambda i: (0, i, 0)),
                  pl.BlockSpec((f, NH), lambda i: (0, 0)),
                  vspec, wspec, vspec, vspec] + [wspec] * nproj,
        out_specs=[spec] * (1 + nproj),
        out_shape=[jax.ShapeDtypeStruct((n, NH), jnp.float32)] * (1 + nproj),
    )(x, w1, mp['b1'].reshape(1, NH), mp['W2'], mp['g'].reshape(1, NH),
      mp['beta'].reshape(1, NH), *proj)
    return out if nproj else out[0]


def _edge_body(ea_ref, gs_ref, gr_ref, w1_ref, b1_ref, w2_ref, g_ref, beta_ref,
               *o_refs, with_res):
    ea = ea_ref[...]
    h = (jnp.dot(ea, w1_ref[...], preferred_element_type=jnp.float32)
         + gs_ref[...] + gr_ref[...] + b1_ref[...])
    h = h * jax.nn.sigmoid(h)
    y = jnp.dot(h, w2_ref[...], preferred_element_type=jnp.float32)
    eu = _layernorm(y, g_ref[...], beta_ref[...])
    o_refs[0][...] = eu
    if with_res:
        o_refs[1][...] = ea + eu


def _edge_call(ea, gs, gr, w1e, ep, with_res):
    spec = pl.BlockSpec((BLK, NH), lambda i: (i, 0))
    wspec = pl.BlockSpec((NH, NH), lambda i: (0, 0))
    vspec = pl.BlockSpec((1, NH), lambda i: (0, 0))
    out_specs = [spec] + ([spec] if with_res else [])
    out_shape = [jax.ShapeDtypeStruct((NE, NH), jnp.float32)]
    if with_res:
        out_shape.append(jax.ShapeDtypeStruct((NE, NH), jnp.float32))
    out = pl.pallas_call(
        functools.partial(_edge_body, with_res=with_res),
        grid=(NE // BLK,),
        in_specs=[spec, spec, spec, wspec, vspec, wspec, vspec, vspec],
        out_specs=out_specs,
        out_shape=out_shape,
    )(ea, gs, gr, w1e, ep['b1'].reshape(1, NH), ep['W2'],
      ep['g'].reshape(1, NH), ep['beta'].reshape(1, NH))
    return out if with_res else (out[0], None)


def _recv_body(xr_ref, sums_ref, cnt_ref, w1u_ref, w1v_ref, b1_ref, w2_ref,
               g_ref, beta_ref, *refs, nproj):
    pw = refs[:nproj]
    o_ref = refs[nproj]
    po = refs[nproj + 1:]
    xr = xr_ref[...]
    cnt = cnt_ref[0]
    c = cnt[0, :, 0:1] + cnt[1, :, 0:1]
    coll = sums_ref[...] / jnp.maximum(c, 1.0)
    h = (jnp.dot(xr, w1u_ref[...], preferred_element_type=jnp.float32)
         + jnp.dot(coll, w1v_ref[...], preferred_element_type=jnp.float32)
         + b1_ref[...])
    h = h * jax.nn.sigmoid(h)
    y = jnp.dot(h, w2_ref[...], preferred_element_type=jnp.float32)
    out = xr + _layernorm(y, g_ref[...], beta_ref[...])
    o_ref[...] = out
    for k in range(nproj):
        po[k][...] = jnp.dot(out, pw[k][...], preferred_element_type=jnp.float32)


def _recv_call(xr, sums, cnt, cset, rp, proj=()):
    spec = pl.BlockSpec((BLK, NH), lambda i: (i, 0))
    wspec = pl.BlockSpec((NH, NH), lambda i: (0, 0))
    vspec = pl.BlockSpec((1, NH), lambda i: (0, 0))
    w1 = rp['W1']
    nproj = len(proj)
    out = pl.pallas_call(
        functools.partial(_recv_body, nproj=nproj),
        grid=(NNODE // BLK,),
        in_specs=[spec, spec,
                  pl.BlockSpec((1, 2, BLK, 16), lambda i: (cset, 0, i, 0)),
                  wspec, wspec, vspec, wspec, vspec, vspec] + [wspec] * nproj,
        out_specs=[spec] * (1 + nproj),
        out_shape=[jax.ShapeDtypeStruct((NNODE, NH), jnp.float32)] * (1 + nproj),
    )(xr, sums, cnt, w1[:NH], w1[NH:], rp['b1'].reshape(1, NH), rp['W2'],
      rp['g'].reshape(1, NH), rp['beta'].reshape(1, NH), *proj)
    return out if nproj else out[0]


@functools.lru_cache(maxsize=None)
def _sc_mesh():
    return plsc.VectorSubcoreMesh(core_axis_name="c", subcore_axis_name="s",
                                  num_cores=2, num_subcores=16)


NW = 32
GCH = 40
G_PER_W = NE // NW
G_NCH = G_PER_W // GCH

S_CH = 128
S_NCHUNK = NE // S_CH
Z_ROWS = 1000
C_PER_CORE = S_NCHUNK // 2


def _gather_body(ps_hbm, pr_hbm, idxs_hbm, idxr_hbm, gs_hbm, gr_hbm,
                 idxs_v, idxr_v, rs_a, rr_a, rs_b, rr_b,
                 sem_sa, sem_ra, sem_sb, sem_rb):
    wid = lax.axis_index("s") * 2 + lax.axis_index("c")
    base = pl.multiple_of(wid * G_PER_W, 8)
    pltpu.sync_copy(idxs_hbm.at[pl.ds(base, G_PER_W)], idxs_v)
    pltpu.sync_copy(idxr_hbm.at[pl.ds(base, G_PER_W)], idxr_v)

    def fire(ci, rs_v, rr_v, sem_s, sem_r):
        o = pl.multiple_of(ci * GCH, 8)
        cs = pltpu.async_copy(ps_hbm.at[idxs_v.at[pl.ds(o, GCH)]], rs_v, sem_s)
        cr = pltpu.async_copy(pr_hbm.at[idxr_v.at[pl.ds(o, GCH)]], rr_v, sem_r)
        return cs, cr

    def drain(ci, rs_v, rr_v, sem_s, sem_r):
        o = pl.multiple_of(ci * GCH, 8)
        pltpu.make_async_copy(ps_hbm.at[idxs_v.at[pl.ds(o, GCH)]], rs_v,
                              sem_s).wait()
        pltpu.make_async_copy(pr_hbm.at[idxr_v.at[pl.ds(o, GCH)]], rr_v,
                              sem_r).wait()
        off = pl.multiple_of(base + ci * GCH, 8)
        pltpu.sync_copy(rs_v, gs_hbm.at[pl.ds(off, GCH)])
        pltpu.sync_copy(rr_v, gr_hbm.at[pl.ds(off, GCH)])

    fire(0, rs_a, rr_a, sem_sa, sem_ra)

    def body(k, carry):
        fire(2 * k + 1, rs_b, rr_b, sem_sb, sem_rb)
        drain(2 * k, rs_a, rr_a, sem_sa, sem_ra)
        fire(2 * k + 2, rs_a, rr_a, sem_sa, sem_ra)
        drain(2 * k + 1, rs_b, rr_b, sem_sb, sem_rb)
        return carry

    lax.fori_loop(0, (G_NCH - 1) // 2, body, 0)
    drain(G_NCH - 1, rs_a, rr_a, sem_sa, sem_ra)


@functools.lru_cache(maxsize=None)
def _gather_kernel():
    return pl.kernel(
        _gather_body,
        out_type=(jax.ShapeDtypeStruct((NE, NH), jnp.float32),) * 2,
        mesh=_sc_mesh(),
        compiler_params=pltpu.CompilerParams(use_tc_tiling_on_sc=False),
        scratch_types=[
            pltpu.VMEM((G_PER_W,), jnp.int32),
            pltpu.VMEM((G_PER_W,), jnp.int32),
            pltpu.VMEM((GCH, NH), jnp.float32),
            pltpu.VMEM((GCH, NH), jnp.float32),
            pltpu.VMEM((GCH, NH), jnp.float32),
            pltpu.VMEM((GCH, NH), jnp.float32),
            pltpu.SemaphoreType.DMA,
            pltpu.SemaphoreType.DMA,
            pltpu.SemaphoreType.DMA,
            pltpu.SemaphoreType.DMA,
        ],
    )


def _gather_call(ps, pr, sg, rg):
    return _gather_kernel()(ps, pr, sg, rg)


def _scatter_body(eu_hbm, idx_hbm, zz_hbm, sums_hbm, idx_v, rows_v, idx_b,
                  rows_b, acc_sh, sem_a, sem_b):
    core = lax.axis_index("c")
    t = lax.axis_index("s")
    nch = jnp.where(t < 2, 79, 78)
    nwb = jnp.where(t < 2, 4, 3)

    for cc in range(2):
        col0 = (core * 2 + cc) * 32

        def zc(j, carry):
            off = pl.multiple_of((t + 16 * j) * Z_ROWS, 8)
            pltpu.sync_copy(zz_hbm, acc_sh.at[pl.ds(off, Z_ROWS)])
            return carry

        lax.fori_loop(0, nwb, zc, 0)
        plsc.subcore_barrier()

        def sfire(k, idx_v, rows_v, sem):
            ch = t + 16 * k
            off = pl.multiple_of(ch * S_CH, 8)
            pltpu.async_copy(idx_hbm.at[pl.ds(off, S_CH)], idx_v, sem)
            pltpu.async_copy(eu_hbm.at[pl.ds(off, S_CH), pl.ds(col0, 32)],
                             rows_v, sem)

        def sdrain(k, idx_v, rows_v, sem):
            ch = t + 16 * k
            off = pl.multiple_of(ch * S_CH, 8)
            pltpu.make_async_copy(idx_hbm.at[pl.ds(off, S_CH)], idx_v,
                                  sem).wait()
            pltpu.make_async_copy(eu_hbm.at[pl.ds(off, S_CH), pl.ds(col0, 32)],
                                  rows_v, sem).wait()
            pltpu.sync_copy(rows_v, acc_sh.at[idx_v], add=True)

        sfire(0, idx_v, rows_v, sem_a)

        def sc_body(k, carry):
            nxt = k + 1

            @pl.when(nxt < nch)
            def _():
                lax.cond(lax.rem(k, 2) == 0,
                         lambda: sfire(nxt, idx_b, rows_b, sem_b),
                         lambda: sfire(nxt, idx_v, rows_v, sem_a))

            lax.cond(lax.rem(k, 2) == 0,
                     lambda: sdrain(k, idx_v, rows_v, sem_a),
                     lambda: sdrain(k, idx_b, rows_b, sem_b))
            return carry

        lax.fori_loop(0, nch, sc_body, 0)
        plsc.subcore_barrier()

        def wb(j, carry):
            off = pl.multiple_of((t + 16 * j) * Z_ROWS, 8)
            pltpu.sync_copy(acc_sh.at[pl.ds(off, Z_ROWS)],
                            sums_hbm.at[pl.ds(off, Z_ROWS), pl.ds(col0, 32)])
            return carry

        lax.fori_loop(0, nwb, wb, 0)
        plsc.subcore_barrier()


@functools.lru_cache(maxsize=None)
def _scatter_kernel():
    return pl.kernel(
        _scatter_body,
        out_type=jax.ShapeDtypeStruct((NNODE, NH), jnp.float32),
        mesh=_sc_mesh(),
        compiler_params=pltpu.CompilerParams(use_tc_tiling_on_sc=False),
        scratch_types=[
            pltpu.VMEM((S_CH,), jnp.int32),
            pltpu.VMEM((S_CH, 32), jnp.float32),
            pltpu.VMEM((S_CH,), jnp.int32),
            pltpu.VMEM((S_CH, 32), jnp.float32),
            pltpu.VMEM_SHARED((NNODE, 32), jnp.float32),
            pltpu.SemaphoreType.DMA,
            pltpu.SemaphoreType.DMA,
        ],
    )


def _scatter_call(eu, rs, zz):
    return _scatter_kernel()(eu, rs, zz)


def _counts_body(i0_hbm, i1_hbm, i2_hbm, ones_hbm, zz_hbm, cnt_hbm, idx_v,
                 ones_v, acc_sh):
    core = lax.axis_index("c")
    t = lax.axis_index("s")
    nch = jnp.where(t < 1, 40, 39)
    nwb = jnp.where(t < 2, 4, 3)

    pltpu.sync_copy(ones_hbm, ones_v)

    for si, idx_hbm in enumerate((i0_hbm, i1_hbm, i2_hbm)):
        def zc(j, carry):
            off = pl.multiple_of((t + 16 * j) * Z_ROWS, 8)
            pltpu.sync_copy(zz_hbm, acc_sh.at[pl.ds(off, Z_ROWS)])
            return carry

        lax.fori_loop(0, nwb, zc, 0)
        plsc.subcore_barrier()

        def cbody(k, carry):
            ch = core * C_PER_CORE + t + 16 * k
            off = pl.multiple_of(ch * S_CH, 8)
            pltpu.sync_copy(idx_hbm.at[pl.ds(off, S_CH)], idx_v)
            pltpu.sync_copy(ones_v, acc_sh.at[idx_v], add=True)
            return carry

        lax.fori_loop(0, nch, cbody, 0)
        plsc.subcore_barrier()

        def wb(j, carry):
            off = pl.multiple_of((t + 16 * j) * Z_ROWS, 8)
            pltpu.sync_copy(acc_sh.at[pl.ds(off, Z_ROWS)],
                            cnt_hbm.at[si, core, pl.ds(off, Z_ROWS)])
            return carry

        lax.fori_loop(0, nwb, wb, 0)
        plsc.subcore_barrier()


@functools.lru_cache(maxsize=None)
def _counts_kernel():
    return pl.kernel(
        _counts_body,
        out_type=jax.ShapeDtypeStruct((3, 2, NNODE, 16), jnp.float32),
        mesh=_sc_mesh(),
        compiler_params=pltpu.CompilerParams(use_tc_tiling_on_sc=False),
        scratch_types=[
            pltpu.VMEM((S_CH,), jnp.int32),
            pltpu.VMEM((S_CH, 16), jnp.float32),
            pltpu.VMEM_SHARED((NNODE, 16), jnp.float32),
        ],
    )


def _counts_call(r0, r1, r2, ones, zz16):
    return _counts_kernel()(r0, r1, r2, ones, zz16)


def _mp_layer(mpp, ea, ps, pr, sg, rg, rs, cnt, cset, xr, with_edge_res,
              zz32, next_proj):
    gs, gr = _gather_call(ps, pr, sg, rg)
    eu, ea2 = _edge_call(ea, gs, gr, mpp['edge']['W1'][:NH], mpp['edge'],
                         with_edge_res)
    sums = _scatter_call(eu, rs, zz32)
    r = _recv_call(xr, sums, cnt, cset, mpp['recv'], proj=next_proj)
    return r, ea2


def kernel(x_grid, x_mesh, g2m_edge_attr, mm_edge_attr, m2g_edge_attr,
           g2m_edge_index, mm_edge_index, m2g_edge_index, params):
    p = params
    xg0 = x_grid
    xm0 = x_mesh
    g2m0 = g2m_edge_attr
    mm0 = mm_edge_attr
    m2g0 = m2g_edge_attr

    enc_w1 = p['encoder']['edge']['W1']
    proc_w1 = [lp['edge']['W1'] for lp in p['processors']]
    dec_w1 = p['decoder']['edge']['W1']

    xg, ps_enc = _mlp_call(xg0, p['embed_grid'], proj=(enc_w1[NH:2 * NH],))
    xm, pr_enc = _mlp_call(xm0, p['embed_mesh'], proj=(enc_w1[2 * NH:],))
    g2m = _mlp_call(g2m0, p['embed_g2m'])
    mm = _mlp_call(mm0, p['embed_mm'])
    m2g = _mlp_call(m2g0, p['embed_m2g'])

    def prep_idx(ei):
        return ei[0], ei[1], ei[1]

    g2m_sg, g2m_rg, g2m_rs = prep_idx(g2m_edge_index)
    mm_sg, mm_rg, mm_rs = prep_idx(mm_edge_index)
    m2g_sg, m2g_rg, m2g_rs = prep_idx(m2g_edge_index)

    zz32 = jnp.zeros((Z_ROWS, 32), jnp.float32)
    zz16 = jnp.zeros((Z_ROWS, 16), jnp.float32)
    ones16 = jnp.ones((S_CH, 16), jnp.float32)

    cnt3 = _counts_call(g2m_rs, mm_rs, m2g_rs, ones16, zz16)

    r, _ = _mp_layer(p['encoder'], g2m, ps_enc, pr_enc, g2m_sg, g2m_rg,
                     g2m_rs, cnt3, 0, xm, False, zz32,
                     (proc_w1[0][NH:2 * NH], proc_w1[0][2 * NH:]))
    xm, ps_c, pr_c = r
    xg2, pr_dec = _mlp_call(xg, p['encoder']['send'], residual=True,
                            proj=(dec_w1[2 * NH:],))
    ps_dec = None
    for i, lp in enumerate(p['processors']):
        if i < 4:
            nxt = (proc_w1[i + 1][NH:2 * NH], proc_w1[i + 1][2 * NH:])
        else:
            nxt = (dec_w1[NH:2 * NH],)
        r, mm = _mp_layer(lp, mm, ps_c, pr_c, mm_sg, mm_rg, mm_rs, cnt3, 1,
                          xm, True, zz32, nxt)
        if i < 4:
            xm, ps_c, pr_c = r
        else:
            xm, ps_dec = r
    xg_out, _ = _mp_layer(p['decoder'], m2g, ps_dec, pr_dec, m2g_sg, m2g_rg,
                          m2g_rs, cnt3, 2, xg2, False, zz32, ())
    return xg_out[None]
